# Initial kernel scaffold; baseline (speedup 1.0000x reference)
import jax
import jax.numpy as jnp
from jax.experimental import pallas as pl


def kernel(outputs, targets, calibs, grid_size):
    raise NotImplementedError("write your pallas kernel here")


# speedup vs baseline: 16.3202x; 16.3202x over previous
import jax
import jax.numpy as jnp
from jax import lax
from jax.experimental import pallas as pl

_PC_X0 = 0.0
_PC_Y0 = -39.68
_VOX_X = 0.16
_VOX_Y = 0.16


def _loss_body(out_ref, rows_ref, ys_ref, xs_ref, win_ref, tv_ref, sums_ref):
    _, C, H, W = out_ref.shape
    M = ys_ref.shape[2]
    W9 = W // 9

    rows = rows_ref[0]
    hio_r = lax.broadcasted_iota(jnp.int32, (H, 2 * M), 0)

    wio = lax.broadcasted_iota(jnp.int32, (1, W), 1)
    wmod = wio % 9
    coordmask = ((wmod >= 1) & (wmod <= 3)).astype(jnp.float32)
    shapemask = ((wmod >= 4) & (wmod <= 6)).astype(jnp.float32)
    anglemask = (wmod >= 7).astype(jnp.float32)

    w2 = lax.broadcasted_iota(jnp.int32, (W, W9), 0)
    j2 = lax.broadcasted_iota(jnp.int32, (W, W9), 1)
    sel48 = (w2 == 9 * j2).astype(jnp.float32)

    rm_col = jnp.any(hio_r == rows, axis=1, keepdims=True).astype(jnp.float32)

    ys = ys_ref[0]
    xs = xs_ref[0]
    win = win_ref[0]
    S = (lax.broadcasted_iota(jnp.int32, (H, M), 0) == ys).astype(jnp.float32)
    T = (lax.broadcasted_iota(jnp.int32, (W, M), 0) == xs).astype(jnp.float32)
    xmod = xs % 9
    is_conf = (xmod == 0).astype(jnp.float32) * win
    is_coord = ((xmod >= 1) & (xmod <= 3)).astype(jnp.float32) * win
    is_shape = ((xmod >= 4) & (xmod <= 6)).astype(jnp.float32) * win
    is_angle = (xmod >= 7).astype(jnp.float32) * win
    hio_m = lax.broadcasted_iota(jnp.int32, (H, M), 0)

    Gg = jnp.float32(0.0)
    Sgm = jnp.float32(0.0)
    nmt = jnp.float32(0.0)
    coord_d = jnp.float32(0.0)
    shape_d = jnp.float32(0.0)
    angle_d = jnp.float32(0.0)
    obj_c = jnp.float32(0.0)
    noobj_c = jnp.float32(0.0)
    coord_c = jnp.float32(0.0)
    shape_c = jnp.float32(0.0)
    angle_c = jnp.float32(0.0)

    for c in range(C):
        xc = out_ref[0, c]
        idx_c = (H * c + hio_r) // 9
        m_c = jnp.any(idx_c == rows, axis=1, keepdims=True).astype(jnp.float32)

        sq = xc * xc
        coord_d += jnp.sum(m_c * jnp.sum(sq * coordmask, axis=1, keepdims=True))
        shape_d += jnp.sum(m_c * jnp.sum(sq * shapemask, axis=1, keepdims=True))
        angle_d += jnp.sum(m_c * jnp.sum(sq * anglemask, axis=1, keepdims=True))

        xg = jnp.dot(xc, sel48, preferred_element_type=jnp.float32)
        pg = jax.nn.sigmoid(xg)
        gmat = -jnp.clip(jnp.log(1.0 - pg), -100.0, None)
        row_g = jnp.sum(gmat, axis=1, keepdims=True)
        Gg += jnp.sum(row_g)
        Sgm += jnp.sum(m_c * row_g)
        nmt += jnp.sum(m_c)

        u = jnp.dot(xc, T, preferred_element_type=jnp.float32)
        v = jnp.sum(S * u, axis=0, keepdims=True)
        t = tv_ref[0, c:c + 1, :]
        idx_pt = (H * c + ys) // 9
        o_pt = (hio_m == idx_pt).astype(jnp.float32)
        pm = jnp.sum(o_pt * rm_col, axis=0, keepdims=True)

        pv = jax.nn.sigmoid(v)
        gpv = -jnp.clip(jnp.log(pv), -100.0, None)
        gnv = -jnp.clip(jnp.log(1.0 - pv), -100.0, None)
        bce_delta = t * (gpv - gnv)
        obj_c += jnp.sum(is_conf * pm * bce_delta)
        noobj_c += jnp.sum(is_conf * (1.0 - pm) * bce_delta)
        mse_delta = (v - t) ** 2 - v * v
        coord_c += jnp.sum(is_coord * pm * mse_delta)
        shape_c += jnp.sum(is_shape * pm * mse_delta)
        angle_c += jnp.sum(is_angle * pm * mse_delta)

    vals = [Gg, Sgm, nmt, coord_d, shape_d, angle_d,
            obj_c, noobj_c, coord_c, shape_c, angle_c]
    lane = lax.broadcasted_iota(jnp.int32, (1, 16), 1)
    out_vec = jnp.zeros((1, 16), jnp.float32)
    for k, s in enumerate(vals):
        out_vec = out_vec + jnp.where(lane == k, s, 0.0)
    sums_ref[0] = out_vec


def kernel(outputs, targets, calibs, grid_size):
    B, C, H, W = outputs.shape
    M = targets.shape[1]

    gs = grid_size.astype(jnp.float32)
    gr_y = H / gs[1]
    gr_x = W / gs[2]
    car = targets[:, :, 0] == 0.0
    xyz1 = jnp.concatenate(
        [targets[:, :, 11:14], jnp.ones((B, M, 1), jnp.float32)], axis=2)
    velo = jnp.einsum('bmk,bjk->bmj', xyz1, calibs)
    y_f = (velo[:, :, 1] - _PC_Y0) / _VOX_Y * gr_y
    x_f = (velo[:, :, 0] - _PC_X0) / _VOX_X * gr_x
    y_idx = jnp.clip(y_f.astype(jnp.int32), 0, H - 1)
    x_idx = jnp.clip(x_f.astype(jnp.int32), 0, W - 1)

    key = y_idx * W + x_idx
    mi = jnp.arange(M)
    eq = (key[:, :, None] == key[:, None, :]) & car[:, :, None] & car[:, None, :]
    later = eq & (mi[None, None, :] > mi[None, :, None])
    win = car & ~jnp.any(later, axis=2)

    rows = jnp.concatenate(
        [jnp.where(car, y_idx, H), jnp.where(car, x_idx, H)], axis=1
    ).astype(jnp.int32)
    ys_s = jnp.where(car, y_idx, -1).astype(jnp.int32)
    xs_s = jnp.where(car, x_idx, -1).astype(jnp.int32)

    tv = jnp.stack([
        jnp.ones((B, M), jnp.float32),
        targets[:, :, 11], targets[:, :, 12], targets[:, :, 13],
        targets[:, :, 8], targets[:, :, 9], targets[:, :, 10],
        jnp.cos(targets[:, :, 14]), jnp.sin(targets[:, :, 14])], axis=1)

    sums = pl.pallas_call(
        _loss_body,
        grid=(B,),
        in_specs=[
            pl.BlockSpec((1, C, H, W), lambda b: (b, 0, 0, 0)),
            pl.BlockSpec((1, 1, 2 * M), lambda b: (b, 0, 0)),
            pl.BlockSpec((1, 1, M), lambda b: (b, 0, 0)),
            pl.BlockSpec((1, 1, M), lambda b: (b, 0, 0)),
            pl.BlockSpec((1, 1, M), lambda b: (b, 0, 0)),
            pl.BlockSpec((1, C, M), lambda b: (b, 0, 0)),
        ],
        out_specs=pl.BlockSpec((1, 1, 16), lambda b: (b, 0, 0)),
        out_shape=jax.ShapeDtypeStruct((B, 1, 16), jnp.float32),
    )(
        outputs,
        rows.reshape(B, 1, 2 * M),
        ys_s.reshape(B, 1, M),
        xs_s.reshape(B, 1, M),
        win.astype(jnp.float32).reshape(B, 1, M),
        tv,
    )

    s = sums.reshape(B, 16)
    Gg, Sgm, nmt = s[:, 0], s[:, 1], s[:, 2]
    coord_d, shape_d, angle_d = s[:, 3], s[:, 4], s[:, 5]
    obj_c, noobj_c = s[:, 6], s[:, 7]
    coord_c, shape_c, angle_c = s[:, 8], s[:, 9], s[:, 10]

    W9 = W // 9
    ln2 = -jnp.log(jnp.float32(0.5))
    n = C * H * W9
    N = B * n
    obj = jnp.sum(Sgm + (C * H - nmt) * W9 * ln2 + obj_c) / N
    noobj = 0.5 * jnp.sum((Gg - Sgm) + nmt * W9 * ln2 + noobj_c) / N
    coord = 5.0 * jnp.sum(coord_d + coord_c) / (3 * N)
    shape_l = 5.0 * jnp.sum(shape_d + shape_c) / (3 * N)
    angle = 5.0 * jnp.sum(angle_d + angle_c) / (2 * N)
    return jnp.stack([obj, noobj, coord, shape_l, angle])
